# baseline (speedup 1.0000x reference)
import jax
import jax.numpy as jnp
import numpy as np
from jax.experimental import pallas as pl
from jax.experimental.pallas import tpu as pltpu

_NUM_BUCKETS = 32
_MAX_DISTANCE = 128
_H = 12
_S = 2048
_TEXT = 1024
_G = 32
_BM = 128


def _bucket_np(relative_position):
    rp = np.asarray(relative_position, dtype=np.int32)
    ret = np.zeros_like(rp)
    n = -rp
    num_buckets = _NUM_BUCKETS // 2
    ret = ret + (n < 0).astype(np.int32) * num_buckets
    n = np.abs(n)
    max_exact = num_buckets // 2
    is_small = n < max_exact
    val_if_large = max_exact + (
        np.log(n.astype(np.float32) / max_exact + 1e-6)
        / np.log(_MAX_DISTANCE / max_exact)
        * (num_buckets - max_exact)
    ).astype(np.int32)
    val_if_large = np.minimum(val_if_large, num_buckets - 1)
    return (ret + np.where(is_small, n, val_if_large)).astype(np.int32)


def _one_hot(idx):
    return (idx[None, :] == np.arange(_NUM_BUCKETS)[:, None]).astype(np.float32)


def _constants():
    tvec = _bucket_np(np.arange(-(_TEXT - 1), _TEXT, dtype=np.int32))
    tvec = np.concatenate([tvec, np.zeros((1,), np.int32)])
    g = np.arange(_G, dtype=np.int32)
    ia32 = _bucket_np(g[None, :] - g[:, None]).reshape(1, _G * _G)
    j = np.arange(_TEXT, dtype=np.int32)
    til = (j[None, :] % _G == g[:, None]).astype(np.float32)
    stri = (j[None, :] // _G == g[:, None]).astype(np.float32)
    return _one_hot(tvec), ia32, til, stri


_OHT, _IA32, _TIL, _STR = _constants()


def _build_kernel(t0_ref, t1_ref, t2_ref, oht_ref, ia32_ref, til_ref, str_ref,
                  td_ref, ae_ref, be_ref, bands_ref):
    hi = jax.lax.Precision.HIGHEST
    td = (jnp.dot(t0_ref[...], oht_ref[...], precision=hi,
                  preferred_element_type=jnp.float32)
          + t1_ref[:, 0:1] + t2_ref[:, 0:1])
    td_ref[...] = td
    ia32 = ia32_ref[...]
    ae32 = jnp.zeros((_H, _G * _G), jnp.float32)
    be32 = jnp.zeros((_H, _G * _G), jnp.float32)
    for c in range(_NUM_BUCKETS):
        sel = ia32 == c
        ae32 = jnp.where(sel, t1_ref[:, c:c + 1], ae32)
        be32 = jnp.where(sel, t2_ref[:, c:c + 1], be32)
    ae32 = ae32 + t0_ref[:, 0:1]
    til = til_ref[...]
    stri = str_ref[...]
    for r in range(_G):
        ae_ref[:, r * _TEXT:(r + 1) * _TEXT] = jnp.dot(
            ae32[:, r * _G:(r + 1) * _G], til, precision=hi,
            preferred_element_type=jnp.float32)
        be_ref[:, r * _TEXT:(r + 1) * _TEXT] = jnp.dot(
            be32[:, r * _G:(r + 1) * _G], stri, precision=hi,
            preferred_element_type=jnp.float32)
    for oidx, o in enumerate((-1, 0, 1)):
        w2 = td[:, 896 + 128 * o:896 + 128 * o + 256]
        for a in range(_BM // 8):
            rows = [w2[:, 127 - 8 * a - r:255 - 8 * a - r] for r in range(8)]
            bands_ref[:, 8 * a:8 * a + 8, 128 * oidx:128 * (oidx + 1)] = (
                jnp.stack(rows, axis=1))


def _fill_kernel(td_ref, ae_ref, be_ref, bands_ref, out_ref):
    pid = pl.program_id(0)
    z = td_ref[:, _TEXT - 1:_TEXT]
    zfill = jnp.broadcast_to(z[:, :, None], (_H, _BM, _TEXT))
    n_text_steps = _TEXT // _BM

    @pl.when(pid < n_text_steps)
    def _text_rows():
        out_ref[0, :, :, _TEXT:] = zfill
        neg = jnp.broadcast_to(td_ref[:, 0:1][:, :, None], (_H, _BM, 128))
        pos = jnp.broadcast_to(td_ref[:, 2046:2047][:, :, None],
                               (_H, _BM, 128))
        for b in range(_TEXT // 128):
            d = b - pid

            @pl.when(d < -1)
            def _a(b=b):
                out_ref[0, :, :, 128 * b:128 * (b + 1)] = neg

            @pl.when(d > 1)
            def _b(b=b):
                out_ref[0, :, :, 128 * b:128 * (b + 1)] = pos

            for oidx in range(3):
                @pl.when(d == oidx - 1)
                def _t(b=b, oidx=oidx):
                    out_ref[0, :, :, 128 * b:128 * (b + 1)] = (
                        bands_ref[:, :, 128 * oidx:128 * (oidx + 1)])

    @pl.when(pid >= n_text_steps)
    def _image_rows():
        out_ref[0, :, :, 0:_TEXT] = zfill
        ae = ae_ref[...]
        for cb in range(_BM // _G):
            out_ref[0, :, cb * _G:(cb + 1) * _G, _TEXT:] = (
                ae + be_ref[:, 0, cb:cb + 1, :])


@jax.jit
def _bias(rel_embedding_0, rel_embedding_1, rel_embedding_2):
    full = lambda shape: pl.BlockSpec(shape, lambda *_: (0,) * len(shape))
    td, ae2d, be2d, bands = pl.pallas_call(
        _build_kernel,
        in_specs=[full((_H, _NUM_BUCKETS))] * 3 + [
            full((_NUM_BUCKETS, _S)), full((1, _G * _G)),
            full((_G, _TEXT)), full((_G, _TEXT))],
        out_specs=[full((_H, _S)), full((_H, _G * _TEXT)),
                   full((_H, _G * _TEXT)), full((_H, _BM, 384))],
        out_shape=[
            jax.ShapeDtypeStruct((_H, _S), jnp.float32),
            jax.ShapeDtypeStruct((_H, _G * _TEXT), jnp.float32),
            jax.ShapeDtypeStruct((_H, _G * _TEXT), jnp.float32),
            jax.ShapeDtypeStruct((_H, _BM, 384), jnp.float32),
        ],
    )(rel_embedding_0, rel_embedding_1, rel_embedding_2,
      jnp.asarray(_OHT), jnp.asarray(_IA32), jnp.asarray(_TIL),
      jnp.asarray(_STR))
    ae = ae2d.reshape(_H, _G, _TEXT)
    be = be2d.reshape(_H, _G // 4, 4, _TEXT)
    n_text_steps = _S // _BM // 2
    return pl.pallas_call(
        _fill_kernel,
        grid=(_S // _BM,),
        in_specs=[
            pl.BlockSpec((_H, _S), lambda i: (0, 0)),
            pl.BlockSpec((_H, _G, _TEXT), lambda i: (0, 0, 0)),
            pl.BlockSpec((_H, 1, _BM // _G, _TEXT),
                         lambda i: (0, jnp.maximum(i - n_text_steps, 0), 0, 0)),
            pl.BlockSpec((_H, _BM, 384), lambda i: (0, 0, 0)),
        ],
        out_specs=pl.BlockSpec((1, _H, _BM, _S), lambda i: (0, 0, i, 0)),
        out_shape=jax.ShapeDtypeStruct((1, _H, _S, _S), jnp.float32),
    )(td, ae, be, bands)


def kernel(rel_embedding_0, rel_embedding_1, rel_embedding_2):
    return _bias(rel_embedding_0, rel_embedding_1, rel_embedding_2)


# speedup vs baseline: 1.0139x; 1.0139x over previous
import jax
import jax.numpy as jnp
import numpy as np
from jax.experimental import pallas as pl
from jax.experimental.pallas import tpu as pltpu

_NUM_BUCKETS = 32
_MAX_DISTANCE = 128
_H = 12
_S = 2048
_TEXT = 1024
_G = 32
_BM = 128


def _bucket_np(relative_position):
    rp = np.asarray(relative_position, dtype=np.int32)
    ret = np.zeros_like(rp)
    n = -rp
    num_buckets = _NUM_BUCKETS // 2
    ret = ret + (n < 0).astype(np.int32) * num_buckets
    n = np.abs(n)
    max_exact = num_buckets // 2
    is_small = n < max_exact
    val_if_large = max_exact + (
        np.log(n.astype(np.float32) / max_exact + 1e-6)
        / np.log(_MAX_DISTANCE / max_exact)
        * (num_buckets - max_exact)
    ).astype(np.int32)
    val_if_large = np.minimum(val_if_large, num_buckets - 1)
    return (ret + np.where(is_small, n, val_if_large)).astype(np.int32)


def _one_hot(idx):
    return (idx[None, :] == np.arange(_NUM_BUCKETS)[:, None]).astype(np.float32)


def _constants():
    tvec = _bucket_np(np.arange(-(_TEXT - 1), _TEXT, dtype=np.int32))
    tvec = np.concatenate([tvec, np.zeros((1,), np.int32)])
    g = np.arange(_G, dtype=np.int32)
    ia32 = _bucket_np(g[None, :] - g[:, None]).reshape(1, _G * _G)
    j = np.arange(_TEXT, dtype=np.int32)
    til = (j[None, :] % _G == g[:, None]).astype(np.float32)
    stri = (j[None, :] // _G == g[:, None]).astype(np.float32)
    return _one_hot(tvec), ia32, til, stri


_OHT, _IA32, _TIL, _STR = _constants()


def _build_kernel(t0_ref, t1_ref, t2_ref, oht_ref, ia32_ref, til_ref, str_ref,
                  td_ref, ae_ref, be_ref, bands_ref):
    hi = jax.lax.Precision.HIGHEST
    td = (jnp.dot(t0_ref[...], oht_ref[...], precision=hi,
                  preferred_element_type=jnp.float32)
          + t1_ref[:, 0:1] + t2_ref[:, 0:1])
    td_ref[...] = td
    ia32 = ia32_ref[...]
    ae32 = jnp.zeros((_H, _G * _G), jnp.float32)
    be32 = jnp.zeros((_H, _G * _G), jnp.float32)
    for c in range(_NUM_BUCKETS):
        sel = ia32 == c
        ae32 = jnp.where(sel, t1_ref[:, c:c + 1], ae32)
        be32 = jnp.where(sel, t2_ref[:, c:c + 1], be32)
    ae32 = ae32 + t0_ref[:, 0:1]
    til = til_ref[...]
    stri = str_ref[...]
    for r in range(_G):
        ae_ref[:, r * _TEXT:(r + 1) * _TEXT] = jnp.dot(
            ae32[:, r * _G:(r + 1) * _G], til, precision=hi,
            preferred_element_type=jnp.float32)
        be_ref[:, r * _TEXT:(r + 1) * _TEXT] = jnp.dot(
            be32[:, r * _G:(r + 1) * _G], stri, precision=hi,
            preferred_element_type=jnp.float32)
    for oidx, o in enumerate((-1, 0, 1)):
        w2 = td[:, 896 + 128 * o:896 + 128 * o + 256]
        for a in range(_BM // 8):
            rows = [w2[:, 127 - 8 * a - r:255 - 8 * a - r] for r in range(8)]
            bands_ref[:, 8 * a:8 * a + 8, 128 * oidx:128 * (oidx + 1)] = (
                jnp.stack(rows, axis=1))


def _fill_kernel(td_ref, ae_ref, be_ref, bands_ref, out_ref):
    pid = pl.program_id(0)
    z = td_ref[:, _TEXT - 1:_TEXT]
    zfill = jnp.broadcast_to(z[:, :, None], (_H, _BM, _TEXT))
    n_text_steps = _TEXT // _BM

    @pl.when(pid < n_text_steps)
    def _text_rows():
        out_ref[0, :, :, _TEXT:] = zfill
        lane = jax.lax.broadcasted_iota(jnp.int32, (1, 1, _TEXT), 2)
        neg = td_ref[:, 0:1]
        pos = td_ref[:, 2046:2047]
        mixed = jnp.where(lane < 128 * pid, neg[:, :, None], pos[:, :, None])
        out_ref[0, :, :, 0:_TEXT] = jnp.broadcast_to(mixed, (_H, _BM, _TEXT))
        for oidx, o in enumerate((-1, 0, 1)):
            @pl.when(jnp.logical_and(pid + o >= 0, pid + o < n_text_steps))
            def _band(oidx=oidx, o=o):
                out_ref[0, :, :, pl.ds(128 * (pid + o), 128)] = (
                    bands_ref[:, :, 128 * oidx:128 * (oidx + 1)])

    @pl.when(pid >= n_text_steps)
    def _image_rows():
        out_ref[0, :, :, 0:_TEXT] = zfill
        ae = ae_ref[...]
        for cb in range(_BM // _G):
            out_ref[0, :, cb * _G:(cb + 1) * _G, _TEXT:] = (
                ae + be_ref[:, 0, cb:cb + 1, :])


@jax.jit
def _bias(rel_embedding_0, rel_embedding_1, rel_embedding_2):
    full = lambda shape: pl.BlockSpec(shape, lambda *_: (0,) * len(shape))
    td, ae2d, be2d, bands = pl.pallas_call(
        _build_kernel,
        in_specs=[full((_H, _NUM_BUCKETS))] * 3 + [
            full((_NUM_BUCKETS, _S)), full((1, _G * _G)),
            full((_G, _TEXT)), full((_G, _TEXT))],
        out_specs=[full((_H, _S)), full((_H, _G * _TEXT)),
                   full((_H, _G * _TEXT)), full((_H, _BM, 384))],
        out_shape=[
            jax.ShapeDtypeStruct((_H, _S), jnp.float32),
            jax.ShapeDtypeStruct((_H, _G * _TEXT), jnp.float32),
            jax.ShapeDtypeStruct((_H, _G * _TEXT), jnp.float32),
            jax.ShapeDtypeStruct((_H, _BM, 384), jnp.float32),
        ],
    )(rel_embedding_0, rel_embedding_1, rel_embedding_2,
      jnp.asarray(_OHT), jnp.asarray(_IA32), jnp.asarray(_TIL),
      jnp.asarray(_STR))
    ae = ae2d.reshape(_H, _G, _TEXT)
    be = be2d.reshape(_H, _G // 4, 4, _TEXT)
    n_text_steps = _S // _BM // 2
    return pl.pallas_call(
        _fill_kernel,
        grid=(_S // _BM,),
        in_specs=[
            pl.BlockSpec((_H, _S), lambda i: (0, 0)),
            pl.BlockSpec((_H, _G, _TEXT), lambda i: (0, 0, 0)),
            pl.BlockSpec((_H, 1, _BM // _G, _TEXT),
                         lambda i: (0, jnp.maximum(i - n_text_steps, 0), 0, 0)),
            pl.BlockSpec((_H, _BM, 384), lambda i: (0, 0, 0)),
        ],
        out_specs=pl.BlockSpec((1, _H, _BM, _S), lambda i: (0, 0, i, 0)),
        out_shape=jax.ShapeDtypeStruct((1, _H, _S, _S), jnp.float32),
    )(td, ae, be, bands)


def kernel(rel_embedding_0, rel_embedding_1, rel_embedding_2):
    return _bias(rel_embedding_0, rel_embedding_1, rel_embedding_2)
